# baseline (speedup 1.0000x reference)
import jax
import jax.numpy as jnp
from jax.experimental import pallas as pl
from jax.experimental.pallas import tpu as pltpu

_BLK = 400


def _layer1_kernel(x_ref, w_ref, adj_ref, o_ref, adjq_ref, h_ref):
    @pl.when(pl.program_id(0) == 0)
    def _():
        h_ref[...] = jnp.dot(
            x_ref[...], w_ref[...], preferred_element_type=jnp.float32
        ).astype(jnp.bfloat16)

    a = adj_ref[...]
    adjq_ref[0] = jnp.round((a - 0.5) * 254.0).astype(jnp.int8)
    o_ref[...] = jax.nn.relu(
        jnp.dot(
            a.astype(jnp.bfloat16), h_ref[...],
            preferred_element_type=jnp.float32,
        )
    )


def _layer_q_kernel(x_ref, w_ref, adjq_ref, o_ref, h_ref, c_ref):
    @pl.when(pl.program_id(0) == 0)
    def _():
        h = jnp.dot(x_ref[...], w_ref[...], preferred_element_type=jnp.float32)
        h_ref[...] = h.astype(jnp.bfloat16)
        c_ref[...] = 0.5 * jnp.sum(h, axis=0, keepdims=True)

    acc = jnp.dot(
        adjq_ref[0], h_ref[...], preferred_element_type=jnp.float32
    )
    o_ref[...] = jax.nn.relu(acc * (1.0 / 254.0) + c_ref[...])


def _gcn_layer1(x, adj, w):
    n, f = x.shape
    h = w.shape[1]
    return pl.pallas_call(
        _layer1_kernel,
        grid=(n // _BLK,),
        in_specs=[
            pl.BlockSpec((n, f), lambda i: (0, 0)),
            pl.BlockSpec((f, h), lambda i: (0, 0)),
            pl.BlockSpec((_BLK, n), lambda i: (i, 0)),
        ],
        out_specs=[
            pl.BlockSpec((_BLK, h), lambda i: (i, 0)),
            pl.BlockSpec((1, _BLK, n), lambda i: (i, 0, 0)),
        ],
        out_shape=[
            jax.ShapeDtypeStruct((n, h), jnp.float32),
            jax.ShapeDtypeStruct((n // _BLK, _BLK, n), jnp.int8),
        ],
        scratch_shapes=[pltpu.VMEM((n, h), jnp.bfloat16)],
    )(x, w, adj)


def _gcn_layer_q(x, adjq, w):
    n, f = x.shape
    h = w.shape[1]
    return pl.pallas_call(
        _layer_q_kernel,
        grid=(n // _BLK,),
        in_specs=[
            pl.BlockSpec((n, f), lambda i: (0, 0)),
            pl.BlockSpec((f, h), lambda i: (0, 0)),
            pl.BlockSpec((1, _BLK, n), lambda i: (i, 0, 0)),
        ],
        out_specs=pl.BlockSpec((_BLK, h), lambda i: (i, 0)),
        out_shape=jax.ShapeDtypeStruct((n, h), jnp.float32),
        scratch_shapes=[
            pltpu.VMEM((n, h), jnp.bfloat16),
            pltpu.VMEM((1, h), jnp.float32),
        ],
    )(x, w, adjq)


def kernel(features, adj_matrix, W_in, W_h0, W_out):
    x, adjq = _gcn_layer1(features, adj_matrix, W_in)
    x = _gcn_layer_q(x, adjq, W_h0)
    return _gcn_layer_q(x, adjq, W_out)


# speedup vs baseline: 1.0134x; 1.0134x over previous
import jax
import jax.numpy as jnp
from jax.experimental import pallas as pl
from jax.experimental.pallas import tpu as pltpu


def _layer1_kernel(x_ref, w_ref, adj_ref, o_ref, adjq_ref, h_ref):
    @pl.when(pl.program_id(0) == 0)
    def _():
        h_ref[...] = jnp.dot(
            x_ref[...], w_ref[...], preferred_element_type=jnp.float32
        ).astype(jnp.bfloat16)

    a = adj_ref[...]
    adjq_ref[...] = jnp.round((a - 0.5) * 254.0).astype(jnp.int8)
    o_ref[...] = jax.nn.relu(
        jnp.dot(
            a.astype(jnp.bfloat16), h_ref[...],
            preferred_element_type=jnp.float32,
        )
    )


def _layer_q_kernel(x_ref, w_ref, adjq_ref, o_ref, h_ref, c_ref):
    @pl.when(pl.program_id(0) == 0)
    def _():
        h = jnp.dot(x_ref[...], w_ref[...], preferred_element_type=jnp.float32)
        h_ref[...] = h.astype(jnp.bfloat16)
        c_ref[...] = 0.5 * jnp.sum(h, axis=0, keepdims=True)

    acc = jnp.dot(
        adjq_ref[...], h_ref[...], preferred_element_type=jnp.float32
    )
    o_ref[...] = jax.nn.relu(acc * (1.0 / 254.0) + c_ref[...])


def _gcn_layer1(x, adj, w, blk):
    n, f = x.shape
    h = w.shape[1]
    return pl.pallas_call(
        _layer1_kernel,
        grid=(n // blk,),
        in_specs=[
            pl.BlockSpec((n, f), lambda i: (0, 0)),
            pl.BlockSpec((f, h), lambda i: (0, 0)),
            pl.BlockSpec((blk, n), lambda i: (i, 0)),
        ],
        out_specs=[
            pl.BlockSpec((blk, h), lambda i: (i, 0)),
            pl.BlockSpec((blk, n), lambda i: (i, 0)),
        ],
        out_shape=[
            jax.ShapeDtypeStruct((n, h), jnp.float32),
            jax.ShapeDtypeStruct((n, n), jnp.int8),
        ],
        scratch_shapes=[pltpu.VMEM((n, h), jnp.bfloat16)],
    )(x, w, adj)


def _gcn_layer_q(x, adjq, w, blk):
    n, f = x.shape
    h = w.shape[1]
    return pl.pallas_call(
        _layer_q_kernel,
        grid=(n // blk,),
        in_specs=[
            pl.BlockSpec((n, f), lambda i: (0, 0)),
            pl.BlockSpec((f, h), lambda i: (0, 0)),
            pl.BlockSpec((blk, n), lambda i: (i, 0)),
        ],
        out_specs=pl.BlockSpec((blk, h), lambda i: (i, 0)),
        out_shape=jax.ShapeDtypeStruct((n, h), jnp.float32),
        scratch_shapes=[
            pltpu.VMEM((n, h), jnp.bfloat16),
            pltpu.VMEM((1, h), jnp.float32),
        ],
    )(x, w, adjq)


def kernel(features, adj_matrix, W_in, W_h0, W_out):
    x, adjq = _gcn_layer1(features, adj_matrix, W_in, 400)
    x = _gcn_layer_q(x, adjq, W_h0, 1000)
    return _gcn_layer_q(x, adjq, W_out, 1000)


# speedup vs baseline: 1.0290x; 1.0154x over previous
import jax
import jax.numpy as jnp
from jax.experimental import pallas as pl
from jax.experimental.pallas import tpu as pltpu

_QBLK = 1000


def _layer1_kernel(x_ref, w_ref, adj_ref, o_ref, adjq_ref, h_ref):
    @pl.when(pl.program_id(0) == 0)
    def _():
        h_ref[...] = jnp.dot(
            x_ref[...], w_ref[...], preferred_element_type=jnp.float32
        ).astype(jnp.bfloat16)

    a = adj_ref[...]
    adjq_ref[...] = jnp.round((a - 0.5) * 254.0).astype(jnp.int8)
    o_ref[...] = jax.nn.relu(
        jnp.dot(
            a.astype(jnp.bfloat16), h_ref[...],
            preferred_element_type=jnp.float32,
        )
    )


def _layer23_kernel(x_ref, w2_ref, w3_ref, adjq_ref, o_ref,
                    h_ref, c_ref, y2_ref):
    l = pl.program_id(0)
    i = pl.program_id(1)

    @pl.when((l == 0) & (i == 0))
    def _():
        h = jnp.dot(x_ref[...], w2_ref[...],
                    preferred_element_type=jnp.float32)
        h_ref[...] = h.astype(jnp.bfloat16)
        c_ref[...] = 0.5 * jnp.sum(h, axis=0, keepdims=True)

    @pl.when((l == 1) & (i == 0))
    def _():
        h = jnp.dot(y2_ref[...], w3_ref[...],
                    preferred_element_type=jnp.float32)
        h_ref[:, : h.shape[1]] = h.astype(jnp.bfloat16)
        c_ref[:, : h.shape[1]] = 0.5 * jnp.sum(h, axis=0, keepdims=True)

    nclass = w3_ref.shape[1]
    acc2 = jnp.dot(adjq_ref[...], h_ref[...],
                   preferred_element_type=jnp.float32)

    @pl.when(l == 0)
    def _():
        y2_ref[pl.ds(i * _QBLK, _QBLK), :] = jax.nn.relu(
            acc2 * (1.0 / 254.0) + c_ref[...]
        )

    @pl.when(l == 1)
    def _():
        o_ref[...] = jax.nn.relu(
            acc2[:, :nclass] * (1.0 / 254.0) + c_ref[:, :nclass]
        )


def _gcn_layer1(x, adj, w, blk):
    n, f = x.shape
    h = w.shape[1]
    return pl.pallas_call(
        _layer1_kernel,
        grid=(n // blk,),
        in_specs=[
            pl.BlockSpec((n, f), lambda i: (0, 0)),
            pl.BlockSpec((f, h), lambda i: (0, 0)),
            pl.BlockSpec((blk, n), lambda i: (i, 0)),
        ],
        out_specs=[
            pl.BlockSpec((blk, h), lambda i: (i, 0)),
            pl.BlockSpec((blk, n), lambda i: (i, 0)),
        ],
        out_shape=[
            jax.ShapeDtypeStruct((n, h), jnp.float32),
            jax.ShapeDtypeStruct((n, n), jnp.int8),
        ],
        scratch_shapes=[pltpu.VMEM((n, h), jnp.bfloat16)],
    )(x, w, adj)


def _gcn_layer23(x, adjq, w2, w3):
    n, f = x.shape
    hid = w2.shape[1]
    nclass = w3.shape[1]
    return pl.pallas_call(
        _layer23_kernel,
        grid=(2, n // _QBLK),
        in_specs=[
            pl.BlockSpec((n, f), lambda l, i: (0, 0)),
            pl.BlockSpec((f, hid), lambda l, i: (0, 0)),
            pl.BlockSpec((hid, nclass), lambda l, i: (0, 0)),
            pl.BlockSpec((_QBLK, n), lambda l, i: (i, 0)),
        ],
        out_specs=pl.BlockSpec((_QBLK, nclass), lambda l, i: (i, 0)),
        out_shape=jax.ShapeDtypeStruct((n, nclass), jnp.float32),
        scratch_shapes=[
            pltpu.VMEM((n, hid), jnp.bfloat16),
            pltpu.VMEM((1, hid), jnp.float32),
            pltpu.VMEM((n, hid), jnp.float32),
        ],
    )(x, w2, w3, adjq)


def kernel(features, adj_matrix, W_in, W_h0, W_out):
    x, adjq = _gcn_layer1(features, adj_matrix, W_in, 400)
    return _gcn_layer23(x, adjq, W_h0, W_out)


# speedup vs baseline: 1.0304x; 1.0014x over previous
import jax
import jax.numpy as jnp
from jax.experimental import pallas as pl
from jax.experimental.pallas import tpu as pltpu

_QBLK = 1000


def _layer1_kernel(x_ref, w_ref, adj_ref, o_ref, adjq_ref, h_ref):
    @pl.when(pl.program_id(0) == 0)
    def _():
        h_ref[...] = jnp.dot(
            x_ref[...], w_ref[...], preferred_element_type=jnp.float32
        ).astype(jnp.bfloat16)

    a = adj_ref[...]
    adjq_ref[...] = jnp.round((a - 0.5) * 254.0).astype(jnp.int8)
    o_ref[...] = jax.nn.relu(
        jnp.dot(
            a.astype(jnp.bfloat16), h_ref[...],
            preferred_element_type=jnp.float32,
        )
    ).astype(jnp.bfloat16)


def _layer23_kernel(x_ref, w2_ref, w3_ref, adjq_ref, o_ref,
                    h_ref, c_ref, y2_ref):
    l = pl.program_id(0)
    i = pl.program_id(1)

    @pl.when((l == 0) & (i == 0))
    def _():
        h = jnp.dot(x_ref[...], w2_ref[...].astype(jnp.bfloat16),
                    preferred_element_type=jnp.float32)
        h_ref[...] = h.astype(jnp.bfloat16)
        c_ref[...] = 0.5 * jnp.sum(h, axis=0, keepdims=True)

    @pl.when((l == 1) & (i == 0))
    def _():
        h = jnp.dot(y2_ref[...], w3_ref[...].astype(jnp.bfloat16),
                    preferred_element_type=jnp.float32)
        h_ref[:, : h.shape[1]] = h.astype(jnp.bfloat16)
        c_ref[:, : h.shape[1]] = 0.5 * jnp.sum(h, axis=0, keepdims=True)

    nclass = w3_ref.shape[1]
    acc2 = jnp.dot(adjq_ref[...], h_ref[...],
                   preferred_element_type=jnp.float32)

    @pl.when(l == 0)
    def _():
        y2_ref[pl.ds(i * _QBLK, _QBLK), :] = jax.nn.relu(
            acc2 * (1.0 / 254.0) + c_ref[...]
        ).astype(jnp.bfloat16)

    @pl.when(l == 1)
    def _():
        o_ref[...] = jax.nn.relu(
            acc2[:, :nclass] * (1.0 / 254.0) + c_ref[:, :nclass]
        )


def _gcn_layer1(x, adj, w, blk):
    n, f = x.shape
    h = w.shape[1]
    return pl.pallas_call(
        _layer1_kernel,
        grid=(n // blk,),
        in_specs=[
            pl.BlockSpec((n, f), lambda i: (0, 0)),
            pl.BlockSpec((f, h), lambda i: (0, 0)),
            pl.BlockSpec((blk, n), lambda i: (i, 0)),
        ],
        out_specs=[
            pl.BlockSpec((blk, h), lambda i: (i, 0)),
            pl.BlockSpec((blk, n), lambda i: (i, 0)),
        ],
        out_shape=[
            jax.ShapeDtypeStruct((n, h), jnp.bfloat16),
            jax.ShapeDtypeStruct((n, n), jnp.int8),
        ],
        scratch_shapes=[pltpu.VMEM((n, h), jnp.bfloat16)],
    )(x, w, adj)


def _gcn_layer23(x, adjq, w2, w3):
    n, f = x.shape
    hid = w2.shape[1]
    nclass = w3.shape[1]
    return pl.pallas_call(
        _layer23_kernel,
        grid=(2, n // _QBLK),
        in_specs=[
            pl.BlockSpec((n, f), lambda l, i: (0, 0)),
            pl.BlockSpec((f, hid), lambda l, i: (0, 0)),
            pl.BlockSpec((hid, nclass), lambda l, i: (0, 0)),
            pl.BlockSpec((_QBLK, n), lambda l, i: (i, 0)),
        ],
        out_specs=pl.BlockSpec((_QBLK, nclass), lambda l, i: (i, 0)),
        out_shape=jax.ShapeDtypeStruct((n, nclass), jnp.float32),
        scratch_shapes=[
            pltpu.VMEM((n, hid), jnp.bfloat16),
            pltpu.VMEM((1, hid), jnp.float32),
            pltpu.VMEM((n, hid), jnp.bfloat16),
        ],
    )(x, w2, w3, adjq)


def kernel(features, adj_matrix, W_in, W_h0, W_out):
    x, adjq = _gcn_layer1(features, adj_matrix, W_in, 400)
    return _gcn_layer23(x, adjq, W_h0, W_out)
